# Initial kernel scaffold; baseline (speedup 1.0000x reference)
import jax
import jax.numpy as jnp
from jax.experimental import pallas as pl


def kernel(data, label, x):
    raise NotImplementedError("write your pallas kernel here")


# speedup vs baseline: 21.1142x; 21.1142x over previous
import functools

import jax
import jax.numpy as jnp
from jax.experimental import pallas as pl
from jax.experimental.pallas import tpu as pltpu

_K = 8
_NCLASS = 100
_BIGP = 3.0e7


def _body(data_ref, xt_ref, lab_ref, out_ref, rs_ref, rp_ref, *, blk, nblk):
    i = pl.program_id(0)

    @pl.when(i == 0)
    def _init():
        rs_ref[...] = jnp.full((_K, 64), jnp.inf, jnp.float32)
        rp_ref[...] = jnp.full((_K, 64), _BIGP, jnp.float32)

    d = data_ref[...]
    rn = jnp.sum(d * d, axis=1, keepdims=True)
    s = rn - 2.0 * jnp.dot(d, xt_ref[...], preferred_element_type=jnp.float32)

    lab = lab_ref[...].reshape(blk, 1).astype(jnp.float32)
    ridx = jax.lax.broadcasted_iota(jnp.int32, (blk, 1), 0) + i * blk
    comb = ridx.astype(jnp.float32) * 128.0 + lab

    bs, bp = [], []
    for _ in range(_K):
        m = jnp.min(s, axis=0, keepdims=True)
        p = jnp.min(jnp.where(s == m, comb, _BIGP), axis=0, keepdims=True)
        s = jnp.where(comb == p, jnp.inf, s)
        bs.append(m)
        bp.append(p)

    cs = jnp.concatenate([rs_ref[...]] + bs, axis=0)
    cp = jnp.concatenate([rp_ref[...]] + bp, axis=0)
    ns, npay = [], []
    for _ in range(_K):
        m = jnp.min(cs, axis=0, keepdims=True)
        p = jnp.min(jnp.where(cs == m, cp, _BIGP), axis=0, keepdims=True)
        cs = jnp.where(cp == p, jnp.inf, cs)
        ns.append(m)
        npay.append(p)
    rs_ref[...] = jnp.concatenate(ns, axis=0)
    rp_ref[...] = jnp.concatenate(npay, axis=0)

    @pl.when(i == nblk - 1)
    def _vote():
        pay = rp_ref[...].astype(jnp.int32)
        lab8 = pay % 128
        cls = jax.lax.broadcasted_iota(jnp.int32, (_NCLASS, 1), 0)
        counts = jnp.zeros((_NCLASS, 64), jnp.int32)
        for j in range(_K):
            counts = counts + (lab8[j : j + 1, :] == cls).astype(jnp.int32)
        mc = jnp.max(counts, axis=0, keepdims=True)
        ans = jnp.min(jnp.where(counts == mc, cls, _NCLASS), axis=0,
                      keepdims=True)
        out_ref[...] = ans


def kernel(data, label, x):
    if x.ndim == 1:
        x = x[None, :]
    n, dim = data.shape
    q = x.shape[0]
    blk = 400
    nblk = n // blk
    assert n % blk == 0 and q == 64 and dim == 128

    out = pl.pallas_call(
        functools.partial(_body, blk=blk, nblk=nblk),
        grid=(nblk,),
        in_specs=[
            pl.BlockSpec((blk, dim), lambda i: (i, 0)),
            pl.BlockSpec((dim, q), lambda i: (0, 0)),
            pl.BlockSpec((1, blk, 1), lambda i: (i, 0, 0)),
        ],
        out_specs=pl.BlockSpec((1, q), lambda i: (0, 0)),
        out_shape=jax.ShapeDtypeStruct((1, q), jnp.int32),
        scratch_shapes=[
            pltpu.VMEM((_K, q), jnp.float32),
            pltpu.VMEM((_K, q), jnp.float32),
        ],
        compiler_params=pltpu.CompilerParams(
            dimension_semantics=("arbitrary",),
        ),
    )(data, x.T, label.reshape(nblk, blk, 1))
    return out.reshape(q, 1)


# speedup vs baseline: 24.9432x; 1.1813x over previous
import functools

import jax
import jax.numpy as jnp
from jax.experimental import pallas as pl
from jax.experimental.pallas import tpu as pltpu

_K = 8
_NCLASS = 100
_BIGP = 3.0e7


def _extract_topk(s, comb, k):
    ms, ps = [], []
    for _ in range(k):
        m = jnp.min(s, axis=0, keepdims=True)
        p = jnp.min(jnp.where(s == m, comb, _BIGP), axis=0, keepdims=True)
        s = jnp.where(comb == p, jnp.inf, s)
        ms.append(m)
        ps.append(p)
    return jnp.concatenate(ms, axis=0), jnp.concatenate(ps, axis=0)


def _body(data_ref, xa_ref, lab_ref, out_ref, rs_ref, rp_ref, *, blk, nblk):
    i = pl.program_id(0)
    half = blk // 2

    @pl.when(i == 0)
    def _init():
        rs_ref[...] = jnp.full((_K, 128), jnp.inf, jnp.float32)
        rp_ref[...] = jnp.full((_K, 128), _BIGP, jnp.float32)

    d = data_ref[...]
    la = jnp.concatenate([d, d * d], axis=1)
    s = jnp.dot(la, xa_ref[...], preferred_element_type=jnp.float32)
    sf = jnp.concatenate([s[:half], s[half:]], axis=1)

    lab = lab_ref[...].reshape(blk, 1).astype(jnp.float32)
    ridx = jax.lax.broadcasted_iota(jnp.int32, (blk, 1), 0) + i * blk
    comb = ridx.astype(jnp.float32) * 128.0 + lab
    combf = jnp.concatenate(
        [jnp.broadcast_to(comb[:half], (half, 64)),
         jnp.broadcast_to(comb[half:], (half, 64))], axis=1)

    bs, bp = _extract_topk(sf, combf, _K)

    cs = jnp.concatenate([rs_ref[...], bs], axis=0)
    cp = jnp.concatenate([rp_ref[...], bp], axis=0)
    ns, npay = _extract_topk(cs, cp, _K)
    rs_ref[...] = ns
    rp_ref[...] = npay

    @pl.when(i == nblk - 1)
    def _finalize():
        fs = jnp.concatenate([ns[:, :64], ns[:, 64:]], axis=0)
        fp = jnp.concatenate([npay[:, :64], npay[:, 64:]], axis=0)
        _, tp = _extract_topk(fs, fp, _K)
        lab8 = tp.astype(jnp.int32) % 128
        cls = jax.lax.broadcasted_iota(jnp.int32, (_NCLASS, 1), 0)
        counts = jnp.zeros((_NCLASS, 64), jnp.int32)
        for j in range(_K):
            counts = counts + (lab8[j : j + 1, :] == cls).astype(jnp.int32)
        mc = jnp.max(counts, axis=0, keepdims=True)
        ans = jnp.min(jnp.where(counts == mc, cls, _NCLASS), axis=0,
                      keepdims=True)
        out_ref[...] = ans


def kernel(data, label, x):
    if x.ndim == 1:
        x = x[None, :]
    n, dim = data.shape
    q = x.shape[0]
    blk = 400
    nblk = n // blk
    assert n % blk == 0 and q == 64 and dim == 128

    xa = jnp.concatenate([-2.0 * x.T, jnp.ones((dim, q), jnp.float32)], axis=0)
    out = pl.pallas_call(
        functools.partial(_body, blk=blk, nblk=nblk),
        grid=(nblk,),
        in_specs=[
            pl.BlockSpec((blk, dim), lambda i: (i, 0)),
            pl.BlockSpec((2 * dim, q), lambda i: (0, 0)),
            pl.BlockSpec((1, blk, 1), lambda i: (i, 0, 0)),
        ],
        out_specs=pl.BlockSpec((1, q), lambda i: (0, 0)),
        out_shape=jax.ShapeDtypeStruct((1, q), jnp.int32),
        scratch_shapes=[
            pltpu.VMEM((_K, 128), jnp.float32),
            pltpu.VMEM((_K, 128), jnp.float32),
        ],
        compiler_params=pltpu.CompilerParams(
            dimension_semantics=("arbitrary",),
        ),
    )(data, xa, label.reshape(nblk, blk, 1))
    return out.reshape(q, 1)
